# baseline (speedup 1.0000x reference)
import functools

import jax
import jax.numpy as jnp
from jax import lax
from jax.experimental import pallas as pl
from jax.experimental.pallas import tpu as pltpu
from jax.experimental.pallas import tpu_sc as plsc

NUM = 100000
RANK = 32
OUT_DIM = 128
BATCH = 4096
FIELDS = 26
BF = BATCH * FIELDS

NC = 2
NS = 16
NW = NC * NS
BPW = BF // NW
NBUF = 3
CH = 256
NCHUNK = BPW // CH

_P_COLS = 10240
_P_GRID = 10
_P_ROWS = _P_COLS * _P_GRID

_sc_mesh = plsc.VectorSubcoreMesh(core_axis_name="c", subcore_axis_name="s")


@functools.partial(
    pl.kernel,
    mesh=_sc_mesh,
    out_type=jax.ShapeDtypeStruct((BF, OUT_DIM), jnp.float32),
    scratch_types=[
        pltpu.VMEM((BPW,), jnp.int32),
        pltpu.VMEM((NBUF, CH, OUT_DIM), jnp.float32),
        pltpu.SemaphoreType.DMA,
        pltpu.SemaphoreType.DMA,
        pltpu.SemaphoreType.DMA,
        pltpu.SemaphoreType.DMA,
        pltpu.SemaphoreType.DMA,
        pltpu.SemaphoreType.DMA,
    ],
)
def _sc_gather(p_hbm, idx_hbm, out_hbm, idx_v, rows_v, g0, g1, g2, s0, s1, s2):
    wid = lax.axis_index("s") * NC + lax.axis_index("c")
    base = wid * BPW
    pltpu.sync_copy(idx_hbm.at[pl.ds(base, BPW)], idx_v)

    gsem = (g0, g1, g2)
    ssem = (s0, s1, s2)

    def start_gather(j):
        b = j % NBUF
        return pltpu.async_copy(
            p_hbm.at[idx_v.at[pl.ds(j * CH, CH)]], rows_v.at[b], gsem[b])

    def start_store(j):
        b = j % NBUF
        return pltpu.async_copy(
            rows_v.at[b], out_hbm.at[pl.ds(base + j * CH, CH)], ssem[b])

    gath = [None] * NBUF
    stor = [None] * NBUF
    gath[0] = start_gather(0)
    for j in range(1, NCHUNK):
        b = j % NBUF
        pb = (j - 1) % NBUF
        if stor[b] is not None:
            stor[b].wait()
        gath[b] = start_gather(j)
        gath[pb].wait()
        stor[pb] = start_store(j - 1)
    lb = (NCHUNK - 1) % NBUF
    gath[lb].wait()
    stor[lb] = start_store(NCHUNK - 1)
    for j in range(NCHUNK - NBUF + 1, NCHUNK + 1):
        stor[j % NBUF].wait()


def _proj_body(t_ref, w_ref, p_ref):
    p_ref[...] = lax.dot_general(
        t_ref[...], w_ref[...],
        (((0,), (0,)), ((), ())),
        preferred_element_type=jnp.float32,
    )


def _tc_project(table_t, w_t):
    return pl.pallas_call(
        _proj_body,
        grid=(_P_GRID,),
        in_specs=[
            pl.BlockSpec((RANK, _P_COLS), lambda i: (0, i)),
            pl.BlockSpec((RANK, OUT_DIM), lambda i: (0, 0)),
        ],
        out_specs=pl.BlockSpec((_P_COLS, OUT_DIM), lambda i: (i, 0)),
        out_shape=jax.ShapeDtypeStruct((_P_ROWS, OUT_DIM), jnp.float32),
    )(table_t, w_t)


def kernel(src, emb_table, W):
    idx = jnp.transpose(src).reshape(-1).astype(jnp.int32)
    proj = _tc_project(jnp.transpose(emb_table), jnp.transpose(W))
    g = _sc_gather(proj, idx)
    return jnp.transpose(g.reshape(FIELDS, BATCH, OUT_DIM), (1, 0, 2))


# speedup vs baseline: 1.0299x; 1.0299x over previous
import functools

import jax
import jax.numpy as jnp
from jax import lax
from jax.experimental import pallas as pl
from jax.experimental.pallas import tpu as pltpu
from jax.experimental.pallas import tpu_sc as plsc

NUM = 100000
RANK = 32
OUT_DIM = 128
BATCH = 4096
FIELDS = 26
BF = BATCH * FIELDS

NC = 2
NS = 16
NW = NC * NS
BPW = BF // NW
NBUF = 2
CH = 416
NCHUNK = BPW // CH

_P_COLS = 20480
_P_GRID = 5
_P_ROWS = _P_COLS * _P_GRID

_sc_mesh = plsc.VectorSubcoreMesh(core_axis_name="c", subcore_axis_name="s")


@functools.partial(
    pl.kernel,
    mesh=_sc_mesh,
    out_type=jax.ShapeDtypeStruct((BF, OUT_DIM), jnp.float32),
    scratch_types=[
        pltpu.VMEM((BPW,), jnp.int32),
        pltpu.VMEM((NBUF, CH, OUT_DIM), jnp.float32),
        pltpu.SemaphoreType.DMA,
        pltpu.SemaphoreType.DMA,
        pltpu.SemaphoreType.DMA,
        pltpu.SemaphoreType.DMA,
    ],
)
def _sc_gather(p_hbm, idx_hbm, out_hbm, idx_v, rows_v, g0, g1, s0, s1):
    wid = lax.axis_index("s") * NC + lax.axis_index("c")
    base = wid * BPW
    pltpu.sync_copy(idx_hbm.at[pl.ds(base, BPW)], idx_v)

    gsem = (g0, g1)
    ssem = (s0, s1)

    def start_gather(j):
        b = j % NBUF
        return pltpu.async_copy(
            p_hbm.at[idx_v.at[pl.ds(j * CH, CH)]], rows_v.at[b], gsem[b])

    def start_store(j):
        b = j % NBUF
        return pltpu.async_copy(
            rows_v.at[b], out_hbm.at[pl.ds(base + j * CH, CH)], ssem[b])

    gath = [None] * NBUF
    stor = [None] * NBUF
    gath[0] = start_gather(0)
    for j in range(1, NCHUNK):
        b = j % NBUF
        pb = (j - 1) % NBUF
        if stor[b] is not None:
            stor[b].wait()
        gath[b] = start_gather(j)
        gath[pb].wait()
        stor[pb] = start_store(j - 1)
    lb = (NCHUNK - 1) % NBUF
    gath[lb].wait()
    stor[lb] = start_store(NCHUNK - 1)
    for j in range(NCHUNK - NBUF + 1, NCHUNK + 1):
        stor[j % NBUF].wait()


def _proj_body(t_ref, w_ref, p_ref):
    p_ref[...] = lax.dot_general(
        t_ref[...], w_ref[...],
        (((0,), (0,)), ((), ())),
        preferred_element_type=jnp.float32,
    )


def _tc_project(table_t, w_t):
    return pl.pallas_call(
        _proj_body,
        grid=(_P_GRID,),
        in_specs=[
            pl.BlockSpec((RANK, _P_COLS), lambda i: (0, i)),
            pl.BlockSpec((RANK, OUT_DIM), lambda i: (0, 0)),
        ],
        out_specs=pl.BlockSpec((_P_COLS, OUT_DIM), lambda i: (i, 0)),
        out_shape=jax.ShapeDtypeStruct((_P_ROWS, OUT_DIM), jnp.float32),
    )(table_t, w_t)


def kernel(src, emb_table, W):
    idx = jnp.transpose(src).reshape(-1).astype(jnp.int32)
    proj = _tc_project(jnp.transpose(emb_table), jnp.transpose(W))
    g = _sc_gather(proj, idx)
    return jnp.transpose(g.reshape(FIELDS, BATCH, OUT_DIM), (1, 0, 2))
